# Initial kernel scaffold; baseline (speedup 1.0000x reference)
import jax
import jax.numpy as jnp
from jax.experimental import pallas as pl


def kernel(pos, v, z, Wr, Wmix, w_embed, w_vinit, w_out):
    raise NotImplementedError("write your pallas kernel here")


# speedup vs baseline: 14.8811x; 14.8811x over previous
import functools

import jax
import jax.numpy as jnp
from jax import lax
from jax.experimental import pallas as pl
from jax.experimental.pallas import tpu as pltpu
from jax.experimental.pallas import tpu_sc as plsc

_B = 1024
_N = 20
_NF = 16
_NL = 3
_NC = 2
_NS = 16
_NW = _NC * _NS
_GPW = _B // _NW
_BN = _B * _N
_L = 16

_CENTERS = [4.0 * k / 15.0 for k in range(16)]


def _rsqrt(s):
    i = lax.bitcast_convert_type(s, jnp.int32)
    i = jnp.int32(0x5F3759DF) - lax.shift_right_arithmetic(i, 1)
    y = lax.bitcast_convert_type(i, jnp.float32)
    for _ in range(3):
        y = y * (1.5 - 0.5 * s * y * y)
    return y


def _vb(vec, k):
    return vec.at[jnp.full((_L,), k, jnp.int32)].get(mode="promise_in_bounds")


def _tfn_body(nodes_h, wr_h, wmix_h, smalls_h, out_h,
              nodes_v, wr_v, wmix_v, smalls_v,
              rbf_v, geo_v, f0_v, f1_v, a0_v, a1_v, out_v):
    wid = lax.axis_index("s") * _NC + lax.axis_index("c")
    base_node = wid * (_GPW * _N)

    pltpu.sync_copy(nodes_h.at[pl.ds(base_node, _GPW * _N)], nodes_v)
    pltpu.sync_copy(wr_h, wr_v)
    pltpu.sync_copy(wmix_h, wmix_v)
    pltpu.sync_copy(smalls_h, smalls_v)

    iota16 = lax.iota(jnp.int32, _L)
    mask3 = iota16 < 3
    col012 = jnp.where(mask3, iota16, 0)
    ck = [jnp.full((_L,), k, jnp.int32) for k in range(16)]
    z16 = jnp.zeros((_L,), jnp.float32)

    emb = smalls_v[0]
    vini = smalls_v[1]
    wout = smalls_v[2]

    def graph_body(gl, _):
        nb = gl * _N

        def s1_body(b, _c):
            e = b * _L + iota16
            iv = lax.shift_right_logical(e * 3277, 16)
            jv = e - iv * 20
            src = nb + iv
            dst = nb + jv
            xi = plsc.load_gather(nodes_v, [src, ck[0]])
            yi = plsc.load_gather(nodes_v, [src, ck[1]])
            zi = plsc.load_gather(nodes_v, [src, ck[2]])
            xj = plsc.load_gather(nodes_v, [dst, ck[0]])
            yj = plsc.load_gather(nodes_v, [dst, ck[1]])
            zj = plsc.load_gather(nodes_v, [dst, ck[2]])
            dx = xj - xi
            dy = yj - yi
            dz = zj - zi
            s = dx * dx + dy * dy + dz * dz + 1e-8
            y = _rsqrt(s)
            r = s * y
            selfm = iv == jv
            for k in range(16):
                t = r - _CENTERS[k]
                rb = jnp.exp(-2.0 * t * t)
                rb = jnp.where(selfm, 0.0, rb)
                plsc.store_scatter(rbf_v, [e, ck[k]], rb)
            plsc.store_scatter(geo_v, [e, ck[0]], dx * y)
            plsc.store_scatter(geo_v, [e, ck[1]], dy * y)
            plsc.store_scatter(geo_v, [e, ck[2]], dz * y)
            return 0

        lax.fori_loop(0, (_N * _N) // _L, s1_body, 0)

        def init_body(j, _c):
            rr = nb + j
            rspl = jnp.full((_L,), rr, jnp.int32)
            zspl = plsc.load_gather(nodes_v, [rspl, ck[6]])
            f0_v[j] = zspl * emb
            for d in range(3):
                vspl = plsc.load_gather(nodes_v, [rspl, ck[3 + d]])
                f1_v[j * 3 + d] = vspl * vini
            return 0

        lax.fori_loop(0, _N, init_body, 0)

        for l in range(_NL):
            def zero_a0(j, _c):
                a0_v[j] = z16
                return 0

            def zero_a1(t, _c):
                a1_v[t] = z16
                return 0

            lax.fori_loop(0, _N, zero_a0, 0)
            lax.fori_loop(0, _N * 3, zero_a1, 0)

            w0rows = [wr_v[(l * 3 + 0) * 16 + k] for k in range(16)]
            w1rows = [wr_v[(l * 3 + 1) * 16 + k] for k in range(16)]
            w2rows = [wr_v[(l * 3 + 2) * 16 + k] for k in range(16)]

            def i_body(i, _c):
                f0s = f0_v[i]
                f1s = [f1_v[i * 3 + d] for d in range(3)]

                def j_body(j, _c2):
                    e = i * 20 + j
                    espl = jnp.full((_L,), e, jnp.int32)
                    acc0 = z16
                    acc1 = z16
                    acc2 = z16
                    for k in range(16):
                        rk = plsc.load_gather(rbf_v, [espl, ck[k]])
                        acc0 = acc0 + rk * w0rows[k]
                        acc1 = acc1 + rk * w1rows[k]
                        acc2 = acc2 + rk * w2rows[k]
                    plsc.addupdate(a0_v.at[j], acc0 * f0s)
                    for d in range(3):
                        dspl = plsc.load_gather(geo_v, [espl, ck[d]])
                        plsc.addupdate(a1_v.at[j * 3 + d], acc1 * f1s[d] + acc2 * dspl)
                    return 0

                lax.fori_loop(0, _N, j_body, 0)
                return 0

            lax.fori_loop(0, _N, i_body, 0)

            def nu_body(j, _c):
                a0j = a0_v[j]
                f0j = f0_v[j]
                w0r = [wmix_v[(l * 4 + 0) * 16 + c] for c in range(16)]
                w1r = [wmix_v[(l * 4 + 1) * 16 + c] for c in range(16)]
                w2r = [wmix_v[(l * 4 + 2) * 16 + c] for c in range(16)]
                w3r = [wmix_v[(l * 4 + 3) * 16 + c] for c in range(16)]
                acc = z16
                for c in range(16):
                    acc = acc + _vb(a0j, c) * w0r[c] + _vb(f0j, c) * w1r[c]
                new0 = jnp.maximum(acc, 0.0)
                news = []
                for d in range(3):
                    a1jd = a1_v[j * 3 + d]
                    f1jd = f1_v[j * 3 + d]
                    accd = z16
                    for c in range(16):
                        accd = accd + _vb(a1jd, c) * w2r[c] + _vb(f1jd, c) * w3r[c]
                    news.append(accd)
                f0_v[j] = new0
                for d in range(3):
                    f1_v[j * 3 + d] = news[d]
                return 0

            lax.fori_loop(0, _N, nu_body, 0)

        def out_body(j, _c):
            rowidx = jnp.full((_L,), j * 3, jnp.int32) + col012
            acc = z16
            for c in range(16):
                fv = plsc.load_gather(f1_v, [rowidx, ck[c]])
                acc = acc + _vb(wout, c) * fv
            rspl = jnp.full((_L,), nb + j, jnp.int32)
            pr = plsc.load_gather(nodes_v, [rspl, col012])
            acc = acc + pr
            plsc.store_scatter(out_v, [rspl, iota16], acc, mask=mask3)
            return 0

        lax.fori_loop(0, _N, out_body, 0)
        return 0

    lax.fori_loop(0, _GPW, graph_body, 0)

    pltpu.sync_copy(out_v, out_h.at[pl.ds(base_node, _GPW * _N)])


@jax.jit
def kernel(pos, v, z, Wr, Wmix, w_embed, w_vinit, w_out):
    zf = z.astype(jnp.float32)
    nodes = jnp.concatenate(
        [pos, v, zf[:, None], jnp.zeros((_BN, 1), jnp.float32)], axis=1)
    wr_flat = Wr.reshape(_NL * 3 * 16, 16)
    wmix_flat = Wmix.reshape(_NL * 4 * 16, 16)
    smalls = jnp.stack([w_embed, w_vinit, w_out])

    mesh = plsc.VectorSubcoreMesh(
        core_axis_name="c", subcore_axis_name="s",
        num_cores=_NC, num_subcores=_NS)
    run = pl.kernel(
        _tfn_body,
        out_type=jax.ShapeDtypeStruct((_BN, 3), jnp.float32),
        mesh=mesh,
        compiler_params=pltpu.CompilerParams(
            needs_layout_passes=False, use_tc_tiling_on_sc=False),
        scratch_types=[
            pltpu.VMEM((_GPW * _N, 8), jnp.float32),
            pltpu.VMEM((_NL * 3 * 16, 16), jnp.float32),
            pltpu.VMEM((_NL * 4 * 16, 16), jnp.float32),
            pltpu.VMEM((3, 16), jnp.float32),
            pltpu.VMEM((_N * _N, 16), jnp.float32),
            pltpu.VMEM((_N * _N, 4), jnp.float32),
            pltpu.VMEM((_N, 16), jnp.float32),
            pltpu.VMEM((_N * 3, 16), jnp.float32),
            pltpu.VMEM((_N, 16), jnp.float32),
            pltpu.VMEM((_N * 3, 16), jnp.float32),
            pltpu.VMEM((_GPW * _N, 3), jnp.float32),
        ],
    )
    return run(nodes, wr_flat, wmix_flat, smalls)


# speedup vs baseline: 25.0528x; 1.6835x over previous
import functools

import jax
import jax.numpy as jnp
from jax import lax
from jax.experimental import pallas as pl
from jax.experimental.pallas import tpu as pltpu
from jax.experimental.pallas import tpu_sc as plsc

_B = 1024
_N = 20
_NF = 16
_NL = 3
_NC = 2
_NS = 16
_NW = _NC * _NS
_GPW = _B // _NW
_BN = _B * _N
_L = 16

_CENTERS = [4.0 * k / 15.0 for k in range(16)]


def _rsqrt(s):
    i = lax.bitcast_convert_type(s, jnp.int32)
    i = jnp.int32(0x5F3759DF) - lax.shift_right_arithmetic(i, 1)
    y = lax.bitcast_convert_type(i, jnp.float32)
    for _ in range(3):
        y = y * (1.5 - 0.5 * s * y * y)
    return y


def _exp_neg(x):
    y = x * 1.4426950408889634
    n = (y - 0.5).astype(jnp.int32)
    nf = n.astype(jnp.float32)
    u = (x - nf * 0.693359375) + nf * 2.1219444005469058e-4
    p = 1.0 / 5040.0
    for c in (1.0 / 720.0, 1.0 / 120.0, 1.0 / 24.0, 1.0 / 6.0, 0.5, 1.0, 1.0):
        p = p * u + c
    scale = lax.bitcast_convert_type(
        lax.shift_left(n + 127, 23), jnp.float32)
    return jnp.where(x > -80.0, p * scale, 0.0)


def _vb(vec, k):
    return vec.at[jnp.full((_L,), k, jnp.int32)].get(mode="promise_in_bounds")


def _tfn_body(nodes_h, wr_h, wmix_h, smalls_h, out_h,
              nodes_v, wr_v, wmix_v, smalls_v,
              rbf_v, geo_v, f0_v, f1_v, a0_v, a1_v, out_v):
    wid = lax.axis_index("s") * _NC + lax.axis_index("c")
    base_node = wid * (_GPW * _N)

    pltpu.sync_copy(nodes_h.at[pl.ds(base_node, _GPW * _N)], nodes_v)
    pltpu.sync_copy(wr_h, wr_v)
    pltpu.sync_copy(wmix_h, wmix_v)
    pltpu.sync_copy(smalls_h, smalls_v)

    iota16 = lax.iota(jnp.int32, _L)
    mask3 = iota16 < 3
    col012 = jnp.where(mask3, iota16, 0)
    ck = [jnp.full((_L,), k, jnp.int32) for k in range(16)]
    z16 = jnp.zeros((_L,), jnp.float32)

    emb = smalls_v[0]
    vini = smalls_v[1]
    wout = smalls_v[2]

    def graph_body(gl, _):
        nb = gl * _N

        def s1_body(b, _c):
            e = b * _L + iota16
            iv = lax.shift_right_logical(e * 3277, 16)
            jv = e - iv * 20
            src = nb + iv
            dst = nb + jv
            xi = plsc.load_gather(nodes_v, [src, ck[0]])
            yi = plsc.load_gather(nodes_v, [src, ck[1]])
            zi = plsc.load_gather(nodes_v, [src, ck[2]])
            xj = plsc.load_gather(nodes_v, [dst, ck[0]])
            yj = plsc.load_gather(nodes_v, [dst, ck[1]])
            zj = plsc.load_gather(nodes_v, [dst, ck[2]])
            dx = xj - xi
            dy = yj - yi
            dz = zj - zi
            s = dx * dx + dy * dy + dz * dz + 1e-8
            y = _rsqrt(s)
            r = s * y
            selfm = iv == jv
            for k in range(16):
                t = r - _CENTERS[k]
                rb = jnp.exp(-2.0 * t * t)
                rb = jnp.where(selfm, 0.0, rb)
                plsc.store_scatter(rbf_v, [e, ck[k]], rb)
            plsc.store_scatter(geo_v, [e, ck[0]], dx * y)
            plsc.store_scatter(geo_v, [e, ck[1]], dy * y)
            plsc.store_scatter(geo_v, [e, ck[2]], dz * y)
            return 0

        lax.fori_loop(0, (_N * _N) // _L, s1_body, 0)

        def init_body(j, _c):
            rr = nb + j
            rspl = jnp.full((_L,), rr, jnp.int32)
            zspl = plsc.load_gather(nodes_v, [rspl, ck[6]])
            f0_v[j] = zspl * emb
            for d in range(3):
                vspl = plsc.load_gather(nodes_v, [rspl, ck[3 + d]])
                f1_v[j * 3 + d] = vspl * vini
            return 0

        lax.fori_loop(0, _N, init_body, 0)

        for l in range(_NL):
            def zero_a0(j, _c):
                a0_v[j] = z16
                return 0

            def zero_a1(t, _c):
                a1_v[t] = z16
                return 0

            lax.fori_loop(0, _N, zero_a0, 0)
            lax.fori_loop(0, _N * 3, zero_a1, 0)

            w0rows = [wr_v[(l * 3 + 0) * 16 + k] for k in range(16)]
            w1rows = [wr_v[(l * 3 + 1) * 16 + k] for k in range(16)]
            w2rows = [wr_v[(l * 3 + 2) * 16 + k] for k in range(16)]

            def i_body(i, _c):
                f0s = f0_v[i]
                f1s = [f1_v[i * 3 + d] for d in range(3)]

                def j_body(j, _c2):
                    e = i * 20 + j
                    espl = jnp.full((_L,), e, jnp.int32)
                    acc0 = z16
                    acc1 = z16
                    acc2 = z16
                    for k in range(16):
                        rk = plsc.load_gather(rbf_v, [espl, ck[k]])
                        acc0 = acc0 + rk * w0rows[k]
                        acc1 = acc1 + rk * w1rows[k]
                        acc2 = acc2 + rk * w2rows[k]
                    f0d = f0_v[j]
                    plsc.addupdate(a0_v.at[j], acc0 * f0s)
                    plsc.addupdate(a0_v.at[i], acc0 * f0d)
                    for d in range(3):
                        dspl = plsc.load_gather(geo_v, [espl, ck[d]])
                        t = acc2 * dspl
                        f1d = f1_v[j * 3 + d]
                        plsc.addupdate(a1_v.at[j * 3 + d], acc1 * f1s[d] + t)
                        plsc.addupdate(a1_v.at[i * 3 + d], acc1 * f1d - t)
                    return 0

                lax.fori_loop(i + 1, _N, j_body, 0)
                return 0

            lax.fori_loop(0, _N, i_body, 0)

            w2r = [wmix_v[(l * 4 + 2) * 16 + c] for c in range(16)]
            w3r = [wmix_v[(l * 4 + 3) * 16 + c] for c in range(16)]

            def nu_body(j, _c):
                jspl = jnp.full((_L,), j, jnp.int32)
                w0r = [wmix_v[(l * 4 + 0) * 16 + c] for c in range(16)]
                w1r = [wmix_v[(l * 4 + 1) * 16 + c] for c in range(16)]
                acc = z16
                for c in range(16):
                    ba = plsc.load_gather(a0_v, [jspl, ck[c]])
                    bf = plsc.load_gather(f0_v, [jspl, ck[c]])
                    acc = acc + ba * w0r[c] + bf * w1r[c]
                new0 = jnp.maximum(acc, 0.0)
                news = []
                for d in range(3):
                    a1jd = a1_v[j * 3 + d]
                    f1jd = f1_v[j * 3 + d]
                    accd = z16
                    for c in range(16):
                        accd = accd + _vb(a1jd, c) * w2r[c] + _vb(f1jd, c) * w3r[c]
                    news.append(accd)
                f0_v[j] = new0
                for d in range(3):
                    f1_v[j * 3 + d] = news[d]
                return 0

            lax.fori_loop(0, _N, nu_body, 0)

        def out_body(j, _c):
            rowidx = jnp.full((_L,), j * 3, jnp.int32) + col012
            acc = z16
            for c in range(16):
                fv = plsc.load_gather(f1_v, [rowidx, ck[c]])
                acc = acc + _vb(wout, c) * fv
            rspl = jnp.full((_L,), nb + j, jnp.int32)
            pr = plsc.load_gather(nodes_v, [rspl, col012])
            acc = acc + pr
            plsc.store_scatter(out_v, [rspl, iota16], acc, mask=mask3)
            return 0

        lax.fori_loop(0, _N, out_body, 0)
        return 0

    lax.fori_loop(0, _GPW, graph_body, 0)

    pltpu.sync_copy(out_v, out_h.at[pl.ds(base_node, _GPW * _N)])


@jax.jit
def kernel(pos, v, z, Wr, Wmix, w_embed, w_vinit, w_out):
    zf = z.astype(jnp.float32)
    nodes = jnp.concatenate(
        [pos, v, zf[:, None], jnp.zeros((_BN, 1), jnp.float32)], axis=1)
    wr_flat = Wr.reshape(_NL * 3 * 16, 16)
    wmix_flat = Wmix.reshape(_NL * 4 * 16, 16)
    smalls = jnp.stack([w_embed, w_vinit, w_out])

    mesh = plsc.VectorSubcoreMesh(
        core_axis_name="c", subcore_axis_name="s",
        num_cores=_NC, num_subcores=_NS)
    run = pl.kernel(
        _tfn_body,
        out_type=jax.ShapeDtypeStruct((_BN, 3), jnp.float32),
        mesh=mesh,
        compiler_params=pltpu.CompilerParams(
            needs_layout_passes=False, use_tc_tiling_on_sc=False),
        scratch_types=[
            pltpu.VMEM((_GPW * _N, 8), jnp.float32),
            pltpu.VMEM((_NL * 3 * 16, 16), jnp.float32),
            pltpu.VMEM((_NL * 4 * 16, 16), jnp.float32),
            pltpu.VMEM((3, 16), jnp.float32),
            pltpu.VMEM((_N * _N, 16), jnp.float32),
            pltpu.VMEM((_N * _N, 4), jnp.float32),
            pltpu.VMEM((_N, 16), jnp.float32),
            pltpu.VMEM((_N * 3, 16), jnp.float32),
            pltpu.VMEM((_N, 16), jnp.float32),
            pltpu.VMEM((_N * 3, 16), jnp.float32),
            pltpu.VMEM((_GPW * _N, 3), jnp.float32),
        ],
    )
    return run(nodes, wr_flat, wmix_flat, smalls)


# speedup vs baseline: 30.7135x; 1.2260x over previous
import functools

import jax
import jax.numpy as jnp
from jax import lax
from jax.experimental import pallas as pl
from jax.experimental.pallas import tpu as pltpu
from jax.experimental.pallas import tpu_sc as plsc

_B = 1024
_N = 20
_NF = 16
_NL = 3
_NC = 2
_NS = 16
_NW = _NC * _NS
_GPW = _B // _NW
_BN = _B * _N
_L = 16

_CENTERS = [4.0 * k / 15.0 for k in range(16)]


def _rsqrt(s):
    i = lax.bitcast_convert_type(s, jnp.int32)
    i = jnp.int32(0x5F3759DF) - lax.shift_right_arithmetic(i, 1)
    y = lax.bitcast_convert_type(i, jnp.float32)
    for _ in range(3):
        y = y * (1.5 - 0.5 * s * y * y)
    return y


def _exp_neg(x):
    y = x * 1.4426950408889634
    n = (y - 0.5).astype(jnp.int32)
    nf = n.astype(jnp.float32)
    u = (x - nf * 0.693359375) + nf * 2.1219444005469058e-4
    p = 1.0 / 5040.0
    for c in (1.0 / 720.0, 1.0 / 120.0, 1.0 / 24.0, 1.0 / 6.0, 0.5, 1.0, 1.0):
        p = p * u + c
    scale = lax.bitcast_convert_type(
        lax.shift_left(n + 127, 23), jnp.float32)
    return jnp.where(x > -80.0, p * scale, 0.0)


def _vb(vec, k):
    return vec.at[jnp.full((_L,), k, jnp.int32)].get(mode="promise_in_bounds")


def _tfn_body(nodes_h, wr_h, wmix_h, smalls_h, out_h,
              nodes_v, wr_v, wmix_v, smalls_v,
              rbf_v, geo_v, f0_v, f1_v, a0_v, a1_v, out_v):
    wid = lax.axis_index("s") * _NC + lax.axis_index("c")
    base_node = wid * (_GPW * _N)

    pltpu.sync_copy(nodes_h.at[pl.ds(base_node, _GPW * _N)], nodes_v)
    pltpu.sync_copy(wr_h, wr_v)
    pltpu.sync_copy(wmix_h, wmix_v)
    pltpu.sync_copy(smalls_h, smalls_v)

    iota16 = lax.iota(jnp.int32, _L)
    mask3 = iota16 < 3
    col012 = jnp.where(mask3, iota16, 0)
    ck = [jnp.full((_L,), k, jnp.int32) for k in range(16)]
    z16 = jnp.zeros((_L,), jnp.float32)

    emb = smalls_v[0]
    vini = smalls_v[1]
    wout = smalls_v[2]

    def graph_body(gl, _):
        nb = gl * _N

        def s1_body(b, _c):
            e = b * _L + iota16
            iv = lax.shift_right_logical(e * 3277, 16)
            jv = e - iv * 20
            src = nb + iv
            dst = nb + jv
            xi = plsc.load_gather(nodes_v, [src, ck[0]])
            yi = plsc.load_gather(nodes_v, [src, ck[1]])
            zi = plsc.load_gather(nodes_v, [src, ck[2]])
            xj = plsc.load_gather(nodes_v, [dst, ck[0]])
            yj = plsc.load_gather(nodes_v, [dst, ck[1]])
            zj = plsc.load_gather(nodes_v, [dst, ck[2]])
            dx = xj - xi
            dy = yj - yi
            dz = zj - zi
            s = dx * dx + dy * dy + dz * dz + 1e-8
            y = _rsqrt(s)
            r = s * y
            selfm = iv == jv
            for k in range(16):
                t = r - _CENTERS[k]
                rb = jnp.exp(-2.0 * t * t)
                rb = jnp.where(selfm, 0.0, rb)
                plsc.store_scatter(rbf_v, [e, ck[k]], rb)
            plsc.store_scatter(geo_v, [e, ck[0]], dx * y)
            plsc.store_scatter(geo_v, [e, ck[1]], dy * y)
            plsc.store_scatter(geo_v, [e, ck[2]], dz * y)
            return 0

        lax.fori_loop(0, (_N * _N) // _L, s1_body, 0)

        def init_body(j, _c):
            rr = nb + j
            rspl = jnp.full((_L,), rr, jnp.int32)
            zspl = plsc.load_gather(nodes_v, [rspl, ck[6]])
            f0_v[j] = zspl * emb
            for d in range(3):
                vspl = plsc.load_gather(nodes_v, [rspl, ck[3 + d]])
                f1_v[j * 3 + d] = vspl * vini
            return 0

        lax.fori_loop(0, _N, init_body, 0)

        for l in range(_NL):
            def zero_a0(j, _c):
                a0_v[j] = z16
                return 0

            def zero_a1(t, _c):
                a1_v[t] = z16
                return 0

            lax.fori_loop(0, _N, zero_a0, 0)
            lax.fori_loop(0, _N * 3, zero_a1, 0)

            w0rows = [wr_v[(l * 3 + 0) * 16 + k] for k in range(16)]
            w1rows = [wr_v[(l * 3 + 1) * 16 + k] for k in range(16)]
            w2rows = [wr_v[(l * 3 + 2) * 16 + k] for k in range(16)]

            def i_body(i, _c):
                f0s = f0_v[i]
                f1s = [f1_v[i * 3 + d] for d in range(3)]

                def j_body(j, car):
                    s0, s1x, s1y, s1z = car
                    e = i * 20 + j
                    espl = jnp.full((_L,), e, jnp.int32)
                    rrow = rbf_v[e]
                    acc0 = z16
                    acc1 = z16
                    acc2 = z16
                    for k in range(16):
                        rk = _vb(rrow, k)
                        acc0 = acc0 + rk * w0rows[k]
                        acc1 = acc1 + rk * w1rows[k]
                        acc2 = acc2 + rk * w2rows[k]
                    f0d = f0_v[j]
                    plsc.addupdate(a0_v.at[j], acc0 * f0s)
                    s0 = s0 + acc0 * f0d
                    snew = []
                    for d, sacc in zip(range(3), (s1x, s1y, s1z)):
                        dspl = plsc.load_gather(geo_v, [espl, ck[d]])
                        t = acc2 * dspl
                        f1d = f1_v[j * 3 + d]
                        plsc.addupdate(a1_v.at[j * 3 + d], acc1 * f1s[d] + t)
                        snew.append(sacc + acc1 * f1d - t)
                    return (s0, snew[0], snew[1], snew[2])

                s0, s1x, s1y, s1z = lax.fori_loop(
                    i + 1, _N, j_body, (z16, z16, z16, z16))
                plsc.addupdate(a0_v.at[i], s0)
                plsc.addupdate(a1_v.at[i * 3 + 0], s1x)
                plsc.addupdate(a1_v.at[i * 3 + 1], s1y)
                plsc.addupdate(a1_v.at[i * 3 + 2], s1z)
                return 0

            lax.fori_loop(0, _N, i_body, 0)

            w2r = [wmix_v[(l * 4 + 2) * 16 + c] for c in range(16)]
            w3r = [wmix_v[(l * 4 + 3) * 16 + c] for c in range(16)]

            def nu_body(j, _c):
                jspl = jnp.full((_L,), j, jnp.int32)
                w0r = [wmix_v[(l * 4 + 0) * 16 + c] for c in range(16)]
                w1r = [wmix_v[(l * 4 + 1) * 16 + c] for c in range(16)]
                acc = z16
                for c in range(16):
                    ba = plsc.load_gather(a0_v, [jspl, ck[c]])
                    bf = plsc.load_gather(f0_v, [jspl, ck[c]])
                    acc = acc + ba * w0r[c] + bf * w1r[c]
                new0 = jnp.maximum(acc, 0.0)
                news = []
                for d in range(3):
                    if d == 0:
                        jd = jnp.full((_L,), j * 3 + d, jnp.int32)
                        ba_ = [plsc.load_gather(a1_v, [jd, ck[c]]) for c in range(16)]
                        bf_ = [plsc.load_gather(f1_v, [jd, ck[c]]) for c in range(16)]
                    else:
                        a1jd = a1_v[j * 3 + d]
                        f1jd = f1_v[j * 3 + d]
                        ba_ = [_vb(a1jd, c) for c in range(16)]
                        bf_ = [_vb(f1jd, c) for c in range(16)]
                    accd = z16
                    for c in range(16):
                        accd = accd + ba_[c] * w2r[c] + bf_[c] * w3r[c]
                    news.append(accd)
                f0_v[j] = new0
                for d in range(3):
                    f1_v[j * 3 + d] = news[d]
                return 0

            lax.fori_loop(0, _N, nu_body, 0)

        def out_body(j, _c):
            rowidx = jnp.full((_L,), j * 3, jnp.int32) + col012
            acc = z16
            for c in range(16):
                fv = plsc.load_gather(f1_v, [rowidx, ck[c]])
                acc = acc + _vb(wout, c) * fv
            rspl = jnp.full((_L,), nb + j, jnp.int32)
            pr = plsc.load_gather(nodes_v, [rspl, col012])
            acc = acc + pr
            plsc.store_scatter(out_v, [rspl, iota16], acc, mask=mask3)
            return 0

        lax.fori_loop(0, _N, out_body, 0)
        return 0

    lax.fori_loop(0, _GPW, graph_body, 0)

    pltpu.sync_copy(out_v, out_h.at[pl.ds(base_node, _GPW * _N)])


@jax.jit
def kernel(pos, v, z, Wr, Wmix, w_embed, w_vinit, w_out):
    zf = z.astype(jnp.float32)
    nodes = jnp.concatenate(
        [pos, v, zf[:, None], jnp.zeros((_BN, 1), jnp.float32)], axis=1)
    wr_flat = Wr.reshape(_NL * 3 * 16, 16)
    wmix_flat = Wmix.reshape(_NL * 4 * 16, 16)
    smalls = jnp.stack([w_embed, w_vinit, w_out])

    mesh = plsc.VectorSubcoreMesh(
        core_axis_name="c", subcore_axis_name="s",
        num_cores=_NC, num_subcores=_NS)
    run = pl.kernel(
        _tfn_body,
        out_type=jax.ShapeDtypeStruct((_BN, 3), jnp.float32),
        mesh=mesh,
        compiler_params=pltpu.CompilerParams(
            needs_layout_passes=False, use_tc_tiling_on_sc=False),
        scratch_types=[
            pltpu.VMEM((_GPW * _N, 8), jnp.float32),
            pltpu.VMEM((_NL * 3 * 16, 16), jnp.float32),
            pltpu.VMEM((_NL * 4 * 16, 16), jnp.float32),
            pltpu.VMEM((3, 16), jnp.float32),
            pltpu.VMEM((_N * _N, 16), jnp.float32),
            pltpu.VMEM((_N * _N, 4), jnp.float32),
            pltpu.VMEM((_N, 16), jnp.float32),
            pltpu.VMEM((_N * 3, 16), jnp.float32),
            pltpu.VMEM((_N, 16), jnp.float32),
            pltpu.VMEM((_N * 3, 16), jnp.float32),
            pltpu.VMEM((_GPW * _N, 3), jnp.float32),
        ],
    )
    return run(nodes, wr_flat, wmix_flat, smalls)
